# speedup vs baseline: 1.8319x; 1.8319x over previous
import functools

import jax
import jax.numpy as jnp
from jax.experimental import pallas as pl
from jax.experimental.pallas import tpu as pltpu


def _gelu_exact(x):
    return 0.5 * x * (1.0 + jax.lax.erf(x * 0.7071067811865476))


def _router_body(x_ref, gw_ref, gates_ref, *, n_experts):
    xb = x_ref[...]
    z = jax.lax.dot_general(
        xb, gw_ref[...], (((1,), (1,)), ((), ())),
        preferred_element_type=jnp.float32)
    idx = jax.lax.broadcasted_iota(jnp.int32, z.shape, 1)
    m1 = jnp.max(z, axis=1, keepdims=True)
    i1 = jnp.min(jnp.where(z == m1, idx, n_experts), axis=1, keepdims=True)
    sel1 = idx == i1
    zm = jnp.where(sel1, -jnp.inf, z)
    m2 = jnp.max(zm, axis=1, keepdims=True)
    i2 = jnp.min(jnp.where(zm == m2, idx, n_experts), axis=1, keepdims=True)
    sel2 = idx == i2
    num = jnp.exp(z - m1)
    denom = 1.0 + jnp.exp(m2 - m1)
    gates_ref[...] = jnp.where(sel1 | sel2, num, 0.0) / denom


def _moe_body(gates_ref, x_ref, w1_ref, b1_ref, w2_ref, b2_ref, out_ref, *,
              block_t, n_experts):
    e = pl.program_id(0)
    t = pl.program_id(1)
    xb = x_ref[...]
    h = jax.lax.dot_general(
        xb, w1_ref[0], (((1,), (1,)), ((), ())),
        preferred_element_type=jnp.float32) + b1_ref[0]
    h = _gelu_exact(h)
    y = jax.lax.dot_general(
        h, w2_ref[0], (((1,), (1,)), ((), ())),
        preferred_element_type=jnp.float32) + b2_ref[0]
    gb = gates_ref[...]
    eidx = jax.lax.broadcasted_iota(jnp.int32, gb.shape, 1)
    ge = jnp.sum(jnp.where(eidx == e, gb, 0.0), axis=1, keepdims=True)
    contrib = ge * y
    rows = pl.ds(t * block_t, block_t)

    @pl.when(e == 0)
    def _():
        out_ref[rows, :] = contrib

    @pl.when(e != 0)
    def _():
        out_ref[rows, :] = out_ref[rows, :] + contrib


def _run(x, gate_w, w1, b1, w2, b2, *, interpret=False):
    n, d = x.shape
    n_experts, h_dim, _ = w1.shape
    block_t = 256 if n % 256 == 0 else n
    n_t = n // block_t

    gates = pl.pallas_call(
        functools.partial(_router_body, n_experts=n_experts),
        grid=(n_t,),
        in_specs=[
            pl.BlockSpec((block_t, d), lambda t: (t, 0)),
            pl.BlockSpec((n_experts, d), lambda t: (0, 0)),
        ],
        out_specs=pl.BlockSpec((block_t, n_experts), lambda t: (t, 0)),
        out_shape=jax.ShapeDtypeStruct((n, n_experts), jnp.float32),
        interpret=interpret,
    )(x, gate_w)

    b1r = b1.reshape(n_experts, 1, h_dim)
    b2r = b2.reshape(n_experts, 1, h_dim)
    out = pl.pallas_call(
        functools.partial(_moe_body, block_t=block_t, n_experts=n_experts),
        grid=(n_experts, n_t),
        in_specs=[
            pl.BlockSpec((block_t, n_experts), lambda e, t: (t, 0)),
            pl.BlockSpec((block_t, d), lambda e, t: (t, 0)),
            pl.BlockSpec((1, h_dim, d), lambda e, t: (e, 0, 0)),
            pl.BlockSpec((1, 1, h_dim), lambda e, t: (e, 0, 0)),
            pl.BlockSpec((1, h_dim, h_dim), lambda e, t: (e, 0, 0)),
            pl.BlockSpec((1, 1, h_dim), lambda e, t: (e, 0, 0)),
        ],
        out_specs=pl.BlockSpec((n, d), lambda e, t: (0, 0)),
        out_shape=jax.ShapeDtypeStruct((n, d), jnp.float32),
        compiler_params=pltpu.CompilerParams(
            dimension_semantics=("arbitrary", "arbitrary")),
        interpret=interpret,
    )(gates, x, w1, b1r, w2, b2r)
    return out


def kernel(hidden_states, gate_w, w1, b1, w2, b2):
    bsz, seq, d = hidden_states.shape
    x = hidden_states.reshape(-1, d)
    out = _run(x, gate_w, w1, b1, w2, b2)
    return out.reshape(bsz, seq, d)
